# baseline (speedup 1.0000x reference)
import functools

import jax
import jax.numpy as jnp
import numpy as np
from jax.experimental import pallas as pl
from jax.experimental.pallas import tpu as pltpu

D = 768
PS = 16
IMG = 224
NPATCH = (IMG // PS) ** 2
PD = PS * PS
NE = 8
NH = 8
DH = D // NH
HID = 256
B = 32
T = B * NPATCH
BM = 784
GM = T // BM
AG = 8


def _lin_kernel(x_ref, w_ref, b_ref, o_ref, *, act):
    y = jnp.dot(x_ref[...], w_ref[...], preferred_element_type=jnp.float32, precision=jax.lax.Precision.HIGHEST)
    y = y + b_ref[...]
    if act == "relu":
        y = jnp.maximum(y, 0.0)
    o_ref[...] = y


def _linear(x, wt, b2d, act=None):
    M, K = x.shape
    N = wt.shape[1]
    rb = b2d.shape[0]
    return pl.pallas_call(
        functools.partial(_lin_kernel, act=act),
        grid=(M // BM,),
        in_specs=[
            pl.BlockSpec((BM, K), lambda i: (i, 0)),
            pl.BlockSpec((K, N), lambda i: (0, 0)),
            pl.BlockSpec((rb, N), lambda i: (0, 0)),
        ],
        out_specs=pl.BlockSpec((BM, N), lambda i: (i, 0)),
        out_shape=jax.ShapeDtypeStruct((M, N), jnp.float32),
    )(x, wt, b2d)


def _attn_kernel(q_ref, kt_ref, v_ref, o_ref, *, scale):
    q = q_ref[0]
    kt = kt_ref[0]
    v = v_ref[0]
    s = jnp.dot(q, kt, preferred_element_type=jnp.float32, precision=jax.lax.Precision.HIGHEST) * scale
    n = AG * B
    ri = jax.lax.broadcasted_iota(jnp.int32, (n, n), 0) // B
    ci = jax.lax.broadcasted_iota(jnp.int32, (n, n), 1) // B
    s = jnp.where(ri == ci, s, -1e30)
    m = jnp.max(s, axis=-1, keepdims=True)
    e = jnp.exp(s - m)
    p = e / jnp.sum(e, axis=-1, keepdims=True)
    o_ref[0] = jnp.dot(p, v, preferred_element_type=jnp.float32, precision=jax.lax.Precision.HIGHEST)


def _attention(q3, kt3, v3):
    n = AG * B
    return pl.pallas_call(
        functools.partial(_attn_kernel, scale=1.0 / np.sqrt(DH)),
        grid=(NPATCH,),
        in_specs=[
            pl.BlockSpec((1, n, DH), lambda i: (i, 0, 0)),
            pl.BlockSpec((1, DH, n), lambda i: (i, 0, 0)),
            pl.BlockSpec((1, n, DH), lambda i: (i, 0, 0)),
        ],
        out_specs=pl.BlockSpec((1, n, DH), lambda i: (i, 0, 0)),
        out_shape=jax.ShapeDtypeStruct((NPATCH, n, DH), jnp.float32),
    )(q3, kt3, v3)


def _experts_kernel(x_ref, gwt_ref, gb_ref, w1t_ref, b1_ref, w2t_ref, b2_ref,
                    o_ref):
    x = x_ref[...]
    logits = jnp.dot(x, gwt_ref[...], preferred_element_type=jnp.float32, precision=jax.lax.Precision.HIGHEST)
    logits = logits + gb_ref[...]
    m = jnp.max(logits, axis=-1, keepdims=True)
    e = jnp.exp(logits - m)
    p = e / jnp.sum(e, axis=-1, keepdims=True)
    idx = jax.lax.broadcasted_iota(jnp.int32, p.shape, 1)
    p1 = jnp.max(p, axis=-1, keepdims=True)
    i1 = jnp.min(jnp.where(p == p1, idx, NE), axis=-1, keepdims=True)
    pm = jnp.where(idx == i1, -jnp.inf, p)
    p2 = jnp.max(pm, axis=-1, keepdims=True)
    i2 = jnp.min(jnp.where(pm == p2, idx, NE), axis=-1, keepdims=True)
    wi = jnp.where((idx == i1) | (idx == i2), p, 0.0) / (p1 + p2)
    acc = jnp.zeros((x.shape[0], D), jnp.float32)
    for i in range(NE):
        h = jnp.dot(x, w1t_ref[i], preferred_element_type=jnp.float32, precision=jax.lax.Precision.HIGHEST)
        h = jnp.maximum(h + b1_ref[i], 0.0)
        eo = jnp.dot(h, w2t_ref[i], preferred_element_type=jnp.float32, precision=jax.lax.Precision.HIGHEST)
        acc = acc + (eo + b2_ref[i]) * wi[:, i:i + 1]
    o_ref[...] = acc


def _experts(x, gwt, gb2d, w1t, b1, w2t, b2):
    return pl.pallas_call(
        _experts_kernel,
        grid=(T // BM,),
        in_specs=[
            pl.BlockSpec((BM, D), lambda i: (i, 0)),
            pl.BlockSpec((D, NE), lambda i: (0, 0)),
            pl.BlockSpec((1, NE), lambda i: (0, 0)),
            pl.BlockSpec((NE, D, HID), lambda i: (0, 0, 0)),
            pl.BlockSpec((NE, 1, HID), lambda i: (0, 0, 0)),
            pl.BlockSpec((NE, HID, D), lambda i: (0, 0, 0)),
            pl.BlockSpec((NE, 1, D), lambda i: (0, 0, 0)),
        ],
        out_specs=pl.BlockSpec((BM, D), lambda i: (i, 0)),
        out_shape=jax.ShapeDtypeStruct((T, D), jnp.float32),
    )(x, gwt, gb2d, w1t, b1, w2t, b2)


def _moe(x2, p):
    xi = _linear(x2, p["inW"].T, p["inb"].reshape(1, D))
    qkv = _linear(xi, p["qkvW"].T, p["qkvb"].reshape(1, 3 * D))
    def to_att(a):
        return (a.reshape(B, NPATCH, NH, DH)
                 .transpose(1, 2, 0, 3)
                 .reshape(NPATCH, NH * B, DH))
    q3 = to_att(qkv[:, :D])
    k3 = to_att(qkv[:, D:2 * D])
    v3 = to_att(qkv[:, 2 * D:])
    o3 = _attention(q3, k3.transpose(0, 2, 1), v3)
    ao = (o3.reshape(NPATCH, NH, B, DH)
            .transpose(2, 0, 1, 3)
            .reshape(T, D))
    xo = _linear(ao, p["oW"].T, p["ob"].reshape(1, D))
    return _experts(xo, p["gW"].T, p["gb"].reshape(1, NE),
                    p["W1"].transpose(0, 2, 1), p["b1"].reshape(NE, 1, HID),
                    p["W2"].transpose(0, 2, 1), p["b2"].reshape(NE, 1, D))


def kernel(x, params):
    n = IMG // PS
    xp = (x.reshape(B, n, PS, n, PS)
           .transpose(0, 1, 3, 2, 4)
           .reshape(T, PD))
    posb = params["pos"].reshape(NPATCH, D) + params["pb"].reshape(1, D)
    posb = jnp.tile(posb, (BM // NPATCH, 1))
    x2 = _linear(xp, params["pW"].T, posb)
    vWt = params["vW"].T
    vb = params["vb"].reshape(1, D)
    cWt = params["cW"].T
    cb = params["cb"].reshape(1, D)
    first = _moe(x2, params["moe1"])
    first_vector = _linear(first, vWt, vb)
    cls_first = _linear(first, cWt, cb)
    second = _moe(first_vector, params["moe2"])
    second_vector = _linear(second, vWt, vb)
    cls_second = _linear(second, cWt, cb)
    sh = (B, NPATCH, D)
    return (first_vector.reshape(sh), second_vector.reshape(sh),
            cls_first.reshape(sh), cls_second.reshape(sh))
